# baseline (speedup 1.0000x reference)
import functools

import jax
import jax.numpy as jnp
from jax.experimental import pallas as pl

REPAIR_P = 0.3
DEGRADE_P = 0.1
CRITICAL_P = 0.01

_R, _C = 4096, 4096
_BLOCK_ROWS = 2048
_BLOCK_COLS = 1024


def _update(eq, rnd, S):
    damaged = eq == 0
    pristine = eq == (S - 1)
    rep_val = jnp.where(rnd < REPAIR_P, jnp.int32(S - 1), jnp.int32(0))
    crit = jnp.logical_and(pristine, rnd < CRITICAL_P)
    nd_val = jnp.where(crit, jnp.int32(0), jnp.where(rnd < DEGRADE_P, eq - 1, eq))
    return jnp.where(damaged, rep_val, nd_val)


def _body(eq_ref, rnd_ref, out_ref, *, S):
    out_ref[...] = _update(eq_ref[...], rnd_ref[...], S)


def kernel(equipment, randomness_source, equipment_states):
    S = equipment_states.shape[0]
    spec = pl.BlockSpec((_BLOCK_ROWS, _BLOCK_COLS), lambda j, i: (i, j))
    return pl.pallas_call(
        functools.partial(_body, S=S),
        grid=(_C // _BLOCK_COLS, _R // _BLOCK_ROWS),
        in_specs=[spec, spec],
        out_specs=spec,
        out_shape=jax.ShapeDtypeStruct((_R, _C), jnp.int32),
    )(equipment, randomness_source)
